# baseline (speedup 1.0000x reference)
import functools

import jax
import jax.numpy as jnp
from jax import lax
from jax.experimental import pallas as pl
from jax.experimental.pallas import tpu as pltpu
from jax.experimental.pallas import tpu_sc as plsc

CODES = 8192
D = 256
BATCH = 16384
COMMIT = 0.25

BM = 512
BN = 1024
GI = BATCH // BM
GJ = CODES // BN

NC = 2
NS = 16
NW = NC * NS
BPW = BATCH // NW
CH = 256


SN = 256
NSUB = BN // SN


def _tc_body(zs_ref, e_ref, idx_ref, loss_ref,
             bestv_ref, besti_ref, sz_ref, se_ref):
    i = pl.program_id(0)
    j = pl.program_id(1)

    @pl.when(j == 0)
    def _init():
        zb = zs_ref[...]
        sz_ref[...] = jnp.sum(zb * zb, axis=1, keepdims=True)
        bestv_ref[...] = jnp.full((BM, 1), jnp.inf, jnp.float32)
        besti_ref[...] = jnp.zeros((BM, 1), jnp.int32)

    @pl.when(i == 0)
    def _cache_se():
        eb = e_ref[...]
        se_ref[0:1, pl.ds(j * BN, BN)] = (
            jnp.sum((eb * eb) * 0.25, axis=1)[None, :])

    sz = sz_ref[...]
    for k in range(NSUB):
        eb = e_ref[pl.ds(k * SN, SN), :]
        se = se_ref[0:1, pl.ds(j * BN + k * SN, SN)]
        dot = lax.dot_general(zs_ref[...], eb, (((1,), (1,)), ((), ())),
                              preferred_element_type=jnp.float32)
        d = (sz + se) + dot
        lv = jnp.min(d, axis=1, keepdims=True)
        ids = lax.broadcasted_iota(jnp.int32, (BM, SN), 1)
        li = (jnp.min(jnp.where(d == lv, ids, SN), axis=1, keepdims=True)
              + (j * BN + k * SN))
        upd = lv < bestv_ref[...]
        bestv_ref[...] = jnp.where(upd, lv, bestv_ref[...])
        besti_ref[...] = jnp.where(upd, li, besti_ref[...])

    @pl.when(j == GJ - 1)
    def _finish():
        idx_ref[...] = besti_ref[...]
        psum = jnp.sum(bestv_ref[...])

        @pl.when(i == 0)
        def _():
            loss_ref[0, 0] = psum

        @pl.when(i > 0)
        def _():
            loss_ref[0, 0] += psum


def _tc_argmin(z_flat, e):
    es = e * -2.0
    return pl.pallas_call(
        _tc_body,
        grid=(GI, GJ),
        in_specs=[
            pl.BlockSpec((BM, D), lambda i, j: (i, 0)),
            pl.BlockSpec((BN, D), lambda i, j: (j, 0)),
        ],
        out_specs=[
            pl.BlockSpec((BM, 1), lambda i, j: (i, 0)),
            pl.BlockSpec(memory_space=pltpu.SMEM),
        ],
        out_shape=[
            jax.ShapeDtypeStruct((BATCH, 1), jnp.int32),
            jax.ShapeDtypeStruct((1, 1), jnp.float32),
        ],
        scratch_shapes=[
            pltpu.VMEM((BM, 1), jnp.float32),
            pltpu.VMEM((BM, 1), jnp.int32),
            pltpu.VMEM((BM, 1), jnp.float32),
            pltpu.VMEM((1, CODES), jnp.float32),
        ],
    )(z_flat, es)


def _sc_gather(table, indices):
    mesh = plsc.VectorSubcoreMesh(
        core_axis_name="c", subcore_axis_name="s",
        num_cores=NC, num_subcores=NS)

    @functools.partial(
        pl.kernel,
        out_type=jax.ShapeDtypeStruct((BATCH, D), jnp.float32),
        mesh=mesh,
        scratch_types=[
            pltpu.VMEM((CH,), jnp.int32),
            pltpu.VMEM((CH, D), jnp.float32),
            pltpu.SemaphoreType.DMA,
        ],
    )
    def gather(table_hbm, idx_hbm, out_hbm, idx_v, rows_v, sem):
        wid = lax.axis_index("s") * NC + lax.axis_index("c")
        base = wid * BPW
        for c in range(BPW // CH):
            off = base + c * CH
            pltpu.sync_copy(idx_hbm.at[pl.ds(off, CH)], idx_v)
            pltpu.async_copy(table_hbm.at[idx_v], rows_v, sem).wait()
            pltpu.sync_copy(rows_v, out_hbm.at[pl.ds(off, CH)])

    return gather(table, indices)


def kernel(z, embedding_weight):
    original_shape = z.shape
    z_flat = z.reshape(-1, D)
    idx2d, loss_sum = _tc_argmin(z_flat, embedding_weight)
    indices = idx2d.reshape(BATCH)
    z_q = _sc_gather(embedding_weight, indices)
    vq_loss = loss_sum[0, 0] * ((1.0 + COMMIT) / float(BATCH * D))
    return (z_q.reshape(original_shape),
            indices.reshape(original_shape[:-1]),
            vq_loss)


# speedup vs baseline: 1.6482x; 1.6482x over previous
import functools

import jax
import jax.numpy as jnp
from jax import lax
from jax.experimental import pallas as pl
from jax.experimental.pallas import tpu as pltpu
from jax.experimental.pallas import tpu_sc as plsc

CODES = 8192
D = 256
BATCH = 16384
COMMIT = 0.25

BM = 512
BN = 1024
GI = BATCH // BM
GJ = CODES // BN

NC = 2
NS = 16
NW = NC * NS
BPW = BATCH // NW
CH = 256


SN = 256
NSUB = BN // SN


LW = 128
NG = SN // LW


def _tc_body(zs_ref, e_ref, idx_ref, loss_ref,
             bestv_ref, bestg_ref, sz_ref, se_ref):
    i = pl.program_id(0)
    j = pl.program_id(1)

    @pl.when(j == 0)
    def _init():
        zb = zs_ref[...]
        sz_ref[...] = jnp.sum(zb * zb, axis=1, keepdims=True)
        bestv_ref[...] = jnp.full((BM, LW), jnp.inf, jnp.float32)
        bestg_ref[...] = jnp.zeros((BM, LW), jnp.int32)

    @pl.when(i == 0)
    def _cache_se():
        eb = e_ref[...]
        se_ref[0:1, pl.ds(j * BN, BN)] = (
            jnp.sum((eb * eb) * 0.25, axis=1)[None, :])

    sz = sz_ref[...]
    for k in range(NSUB):
        eb = e_ref[pl.ds(k * SN, SN), :]
        dot = lax.dot_general(zs_ref[...], eb, (((1,), (1,)), ((), ())),
                              preferred_element_type=jnp.float32)
        for g in range(NG):
            se = se_ref[0:1, pl.ds(j * BN + k * SN + g * LW, LW)]
            d = (sz + se) + dot[:, g * LW:(g + 1) * LW]
            upd = d < bestv_ref[...]
            gid = j * (BN // LW) + k * NG + g
            bestv_ref[...] = jnp.where(upd, d, bestv_ref[...])
            bestg_ref[...] = jnp.where(upd, gid, bestg_ref[...])

    @pl.when(j == GJ - 1)
    def _finish():
        bv = bestv_ref[...]
        vmin = jnp.min(bv, axis=1, keepdims=True)
        lane = lax.broadcasted_iota(jnp.int32, (BM, LW), 1)
        col = bestg_ref[...] * LW + lane
        idx_ref[...] = jnp.min(jnp.where(bv == vmin, col, CODES),
                               axis=1, keepdims=True)
        psum = jnp.sum(vmin)

        @pl.when(i == 0)
        def _():
            loss_ref[0, 0] = psum

        @pl.when(i > 0)
        def _():
            loss_ref[0, 0] += psum


def _tc_argmin(z_flat, e):
    es = e * -2.0
    return pl.pallas_call(
        _tc_body,
        grid=(GI, GJ),
        in_specs=[
            pl.BlockSpec((BM, D), lambda i, j: (i, 0)),
            pl.BlockSpec((BN, D), lambda i, j: (j, 0)),
        ],
        out_specs=[
            pl.BlockSpec((BM, 1), lambda i, j: (i, 0)),
            pl.BlockSpec(memory_space=pltpu.SMEM),
        ],
        out_shape=[
            jax.ShapeDtypeStruct((BATCH, 1), jnp.int32),
            jax.ShapeDtypeStruct((1, 1), jnp.float32),
        ],
        scratch_shapes=[
            pltpu.VMEM((BM, LW), jnp.float32),
            pltpu.VMEM((BM, LW), jnp.int32),
            pltpu.VMEM((BM, 1), jnp.float32),
            pltpu.VMEM((1, CODES), jnp.float32),
        ],
    )(z_flat, es)


def _sc_gather(table, indices):
    mesh = plsc.VectorSubcoreMesh(
        core_axis_name="c", subcore_axis_name="s",
        num_cores=NC, num_subcores=NS)

    @functools.partial(
        pl.kernel,
        out_type=jax.ShapeDtypeStruct((BATCH, D), jnp.float32),
        mesh=mesh,
        scratch_types=[
            pltpu.VMEM((CH,), jnp.int32),
            pltpu.VMEM((CH, D), jnp.float32),
            pltpu.SemaphoreType.DMA,
        ],
    )
    def gather(table_hbm, idx_hbm, out_hbm, idx_v, rows_v, sem):
        wid = lax.axis_index("s") * NC + lax.axis_index("c")
        base = wid * BPW
        for c in range(BPW // CH):
            off = base + c * CH
            pltpu.sync_copy(idx_hbm.at[pl.ds(off, CH)], idx_v)
            pltpu.async_copy(table_hbm.at[idx_v], rows_v, sem).wait()
            pltpu.sync_copy(rows_v, out_hbm.at[pl.ds(off, CH)])

    return gather(table, indices)


def kernel(z, embedding_weight):
    original_shape = z.shape
    z_flat = z.reshape(-1, D)
    idx2d, loss_sum = _tc_argmin(z_flat, embedding_weight)
    indices = idx2d.reshape(BATCH)
    z_q = _sc_gather(embedding_weight, indices)
    vq_loss = loss_sum[0, 0] * ((1.0 + COMMIT) / float(BATCH * D))
    return (z_q.reshape(original_shape),
            indices.reshape(original_shape[:-1]),
            vq_loss)


# speedup vs baseline: 1.6501x; 1.0011x over previous
import functools

import jax
import jax.numpy as jnp
from jax import lax
from jax.experimental import pallas as pl
from jax.experimental.pallas import tpu as pltpu
from jax.experimental.pallas import tpu_sc as plsc

CODES = 8192
D = 256
BATCH = 16384
COMMIT = 0.25

BM = 512
BN = 1024
GI = BATCH // BM
GJ = CODES // BN

NC = 2
NS = 16
NW = NC * NS
BPW = BATCH // NW
CH = 256


SN = 256
NSUB = BN // SN


LW = 128
NG = SN // LW
GB = BM // 8


def _tc_body(zs_ref, e_ref, idx_ref, loss_ref,
             bestv_ref, bestg_ref, sz_ref, se_ref):
    i = pl.program_id(0)
    j = pl.program_id(1)

    @pl.when(j == 0)
    def _init():
        zb = zs_ref[...]
        sz = jnp.sum(zb * zb, axis=1, keepdims=True)
        sz_ref[...] = jnp.broadcast_to(sz, (BM, LW)).reshape(GB, 8, LW)
        bestv_ref[...] = jnp.full((GB, 8, LW), jnp.inf, jnp.float32)
        bestg_ref[...] = jnp.zeros((GB, 8, LW), jnp.int32)

    @pl.when(i == 0)
    def _cache_se():
        eb = e_ref[...]
        se = jnp.sum((eb * eb) * 0.25, axis=1)
        se_ref[:, :, pl.ds(j * BN, BN)] = jnp.broadcast_to(
            se[None, None, :], (1, 8, BN))

    def _dot(k):
        eb = e_ref[pl.ds(k * SN, SN), :]
        return lax.dot_general(zs_ref[...], eb, (((1,), (1,)), ((), ())),
                               preferred_element_type=jnp.float32
                               ).reshape(GB, 8, SN)

    dot_cur = _dot(0)
    for k in range(NSUB):
        dot_nxt = _dot(k + 1) if k + 1 < NSUB else None
        for g in range(NG):
            se = se_ref[:, :, pl.ds(j * BN + k * SN + g * LW, LW)]
            d = (sz_ref[...] + se) + dot_cur[:, :, g * LW:(g + 1) * LW]
            upd = d < bestv_ref[...]
            gid = j * (BN // LW) + k * NG + g
            bestv_ref[...] = jnp.where(upd, d, bestv_ref[...])
            bestg_ref[...] = jnp.where(upd, gid, bestg_ref[...])
        dot_cur = dot_nxt

    @pl.when(j == GJ - 1)
    def _finish():
        bv = bestv_ref[...]
        vmin = jnp.min(bv, axis=2, keepdims=True)
        lane = lax.broadcasted_iota(jnp.int32, (GB, 8, LW), 2)
        col = bestg_ref[...] * LW + lane
        li = jnp.min(jnp.where(bv == vmin, col, CODES),
                     axis=2, keepdims=True)
        idx_ref[...] = li.reshape(BM, 1)
        psum = jnp.sum(vmin)

        @pl.when(i == 0)
        def _():
            loss_ref[0, 0] = psum

        @pl.when(i > 0)
        def _():
            loss_ref[0, 0] += psum


def _tc_argmin(z_flat, e):
    es = e * -2.0
    return pl.pallas_call(
        _tc_body,
        grid=(GI, GJ),
        in_specs=[
            pl.BlockSpec((BM, D), lambda i, j: (i, 0)),
            pl.BlockSpec((BN, D), lambda i, j: (j, 0)),
        ],
        out_specs=[
            pl.BlockSpec((BM, 1), lambda i, j: (i, 0)),
            pl.BlockSpec(memory_space=pltpu.SMEM),
        ],
        out_shape=[
            jax.ShapeDtypeStruct((BATCH, 1), jnp.int32),
            jax.ShapeDtypeStruct((1, 1), jnp.float32),
        ],
        scratch_shapes=[
            pltpu.VMEM((GB, 8, LW), jnp.float32),
            pltpu.VMEM((GB, 8, LW), jnp.int32),
            pltpu.VMEM((GB, 8, LW), jnp.float32),
            pltpu.VMEM((1, 8, CODES), jnp.float32),
        ],
    )(z_flat, es)


def _sc_gather(table, indices):
    mesh = plsc.VectorSubcoreMesh(
        core_axis_name="c", subcore_axis_name="s",
        num_cores=NC, num_subcores=NS)

    @functools.partial(
        pl.kernel,
        out_type=jax.ShapeDtypeStruct((BATCH, D), jnp.float32),
        mesh=mesh,
        scratch_types=[
            pltpu.VMEM((CH,), jnp.int32),
            pltpu.VMEM((CH, D), jnp.float32),
            pltpu.SemaphoreType.DMA,
        ],
    )
    def gather(table_hbm, idx_hbm, out_hbm, idx_v, rows_v, sem):
        wid = lax.axis_index("s") * NC + lax.axis_index("c")
        base = wid * BPW
        for c in range(BPW // CH):
            off = base + c * CH
            pltpu.sync_copy(idx_hbm.at[pl.ds(off, CH)], idx_v)
            pltpu.async_copy(table_hbm.at[idx_v], rows_v, sem).wait()
            pltpu.sync_copy(rows_v, out_hbm.at[pl.ds(off, CH)])

    return gather(table, indices)


def kernel(z, embedding_weight):
    original_shape = z.shape
    z_flat = z.reshape(-1, D)
    idx2d, loss_sum = _tc_argmin(z_flat, embedding_weight)
    indices = idx2d.reshape(BATCH)
    z_q = _sc_gather(embedding_weight, indices)
    vq_loss = loss_sum[0, 0] * ((1.0 + COMMIT) / float(BATCH * D))
    return (z_q.reshape(original_shape),
            indices.reshape(original_shape[:-1]),
            vq_loss)
